# baseline (speedup 1.0000x reference)
import jax
import jax.numpy as jnp
from jax.experimental import pallas as pl
from jax.experimental.pallas import tpu as pltpu

L = 128
H = 512
BSZ = 16
N = 20
TOPK = 5


def _scores_kernel(q_ref, ka_ref, kb_ref, out_ref):
    n = pl.program_id(0)

    @pl.when(n == 0)
    def _():
        out_ref[...] = jnp.full((BSZ, N), -jnp.inf, dtype=jnp.float32)

    k3a = ka_ref[0, 0].reshape(BSZ // 2, L, H)
    k3b = kb_ref[0, 0].reshape(BSZ // 2, L, H)
    batch = jax.lax.broadcasted_iota(jnp.int32, (BSZ, 1), 0)
    acc = jnp.full((BSZ, 1), -jnp.inf, dtype=jnp.float32)
    for b in range(BSZ // 2):
        att = jax.lax.dot_general(
            k3a[b], q_ref[:, b, :], (((1,), (1,)), ((), ())),
            preferred_element_type=jnp.float32)
        acc = jnp.where(batch == b, jnp.max(att), acc)
    for b in range(BSZ // 2):
        att = jax.lax.dot_general(
            k3b[b], q_ref[:, BSZ // 2 + b, :], (((1,), (1,)), ((), ())),
            preferred_element_type=jnp.float32)
        acc = jnp.where(batch == BSZ // 2 + b, jnp.max(att), acc)
    slot = jax.lax.broadcasted_iota(jnp.int32, (BSZ, N), 1)
    out_ref[...] = jnp.where(slot == n, acc, out_ref[...])


def _topk_kernel(s_ref, out_ref):
    s = s_ref[...]
    col = jax.lax.broadcasted_iota(jnp.int32, (BSZ, N), 1)
    for k in range(TOPK):
        m = jnp.max(s, axis=1, keepdims=True)
        hit = jnp.where(s == m, col, N)
        idx = jnp.min(hit, axis=1, keepdims=True)
        out_ref[:, k:k + 1] = idx.astype(jnp.int32)
        s = jnp.where(col == idx, -jnp.inf, s)


def kernel(query, keys, values):
    del values
    q3 = query.reshape(L, BSZ, H)
    keys_s = keys.reshape(N, 2, BSZ // 2, L * H)

    scores = pl.pallas_call(
        _scores_kernel,
        grid=(N,),
        in_specs=[
            pl.BlockSpec((L, BSZ, H), lambda n: (0, 0, 0)),
            pl.BlockSpec((1, 1, BSZ // 2, L * H), lambda n: (n, 0, 0, 0)),
            pl.BlockSpec((1, 1, BSZ // 2, L * H), lambda n: (n, 1, 0, 0)),
        ],
        out_specs=pl.BlockSpec((BSZ, N), lambda n: (0, 0)),
        out_shape=jax.ShapeDtypeStruct((BSZ, N), jnp.float32),
    )(q3, keys_s, keys_s)

    topk_bk = pl.pallas_call(
        _topk_kernel,
        in_specs=[pl.BlockSpec((BSZ, N), lambda: (0, 0))],
        out_specs=pl.BlockSpec((BSZ, TOPK), lambda: (0, 0)),
        out_shape=jax.ShapeDtypeStruct((BSZ, TOPK), jnp.int32),
    )(scores)

    return (scores.reshape(BSZ, 1, N), topk_bk.T)


# speedup vs baseline: 1.0172x; 1.0172x over previous
import jax
import jax.numpy as jnp
from jax.experimental import pallas as pl
from jax.experimental.pallas import tpu as pltpu

L = 128
H = 512
BSZ = 16
N = 20
TOPK = 5


def _scores_kernel(q_ref, ka_ref, kb_ref, out_ref):
    g = pl.program_id(0)

    @pl.when(g == 0)
    def _():
        out_ref[...] = jnp.full((BSZ, N), -jnp.inf, dtype=jnp.float32)

    batch = jax.lax.broadcasted_iota(jnp.int32, (BSZ, 1), 0)
    slot = jax.lax.broadcasted_iota(jnp.int32, (BSZ, N), 1)
    for half, k_ref in ((0, ka_ref), (1, kb_ref)):
        k3 = k_ref[0].reshape(BSZ, L, H)
        acc = jnp.full((BSZ, 1), -jnp.inf, dtype=jnp.float32)
        for b in range(BSZ):
            att = jax.lax.dot_general(
                k3[b], q_ref[:, b, :], (((1,), (1,)), ((), ())),
                preferred_element_type=jnp.float32)
            acc = jnp.where(batch == b, jnp.max(att), acc)
        out_ref[...] = jnp.where(slot == 2 * g + half, acc, out_ref[...])


def _topk_kernel(s_ref, out_ref):
    s = s_ref[...]
    col = jax.lax.broadcasted_iota(jnp.int32, (BSZ, N), 1)
    for k in range(TOPK):
        m = jnp.max(s, axis=1, keepdims=True)
        hit = jnp.where(s == m, col, N)
        idx = jnp.min(hit, axis=1, keepdims=True)
        out_ref[:, k:k + 1] = idx.astype(jnp.int32)
        s = jnp.where(col == idx, -jnp.inf, s)


def kernel(query, keys, values):
    del values
    q3 = query.reshape(L, BSZ, H)

    scores = pl.pallas_call(
        _scores_kernel,
        grid=(N // 2,),
        in_specs=[
            pl.BlockSpec((L, BSZ, H), lambda g: (0, 0, 0)),
            pl.BlockSpec((1, BSZ, L * H), lambda g: (2 * g, 0, 0)),
            pl.BlockSpec((1, BSZ, L * H), lambda g: (2 * g + 1, 0, 0)),
        ],
        out_specs=pl.BlockSpec((BSZ, N), lambda g: (0, 0)),
        out_shape=jax.ShapeDtypeStruct((BSZ, N), jnp.float32),
    )(q3, keys, keys)

    topk_bk = pl.pallas_call(
        _topk_kernel,
        in_specs=[pl.BlockSpec((BSZ, N), lambda: (0, 0))],
        out_specs=pl.BlockSpec((BSZ, TOPK), lambda: (0, 0)),
        out_shape=jax.ShapeDtypeStruct((BSZ, TOPK), jnp.int32),
    )(scores)

    return (scores.reshape(BSZ, 1, N), topk_bk.T)
